# Initial kernel scaffold; baseline (speedup 1.0000x reference)
import jax
import jax.numpy as jnp
from jax import lax
from jax.experimental import pallas as pl
from jax.experimental.pallas import tpu as pltpu
from jax.experimental.pallas import tpu_sc as plsc

_MAX_Z = 100
_CUTOFF2 = 10.0 * 10.0
_DISTANCE_SCALE = 1e-10 * 18897300000.0
_ENERGY_SCALE = 1.602176634e-19
_OUT_SCALE = 2.30707755e-19 / _ENERGY_SCALE

_N = 10000
_NC = 2
_NS = 16
_NW = _NC * _NS
_L = 16
_ROWS_PER_W = (_N + _NW - 1) // _NW


def _zbl_body(px_hbm, py_hbm, pz_hbm, zi_hbm, bt_hbm, ainv_hbm, st_hbm,
              en_hbm, out_hbm, px_v, py_v, pz_v, zi_v, bt_v, ainv_v, st_v,
              en_v, acc_v):
    cid = lax.axis_index("c")
    sid = lax.axis_index("s")
    wid = sid * _NC + cid

    pltpu.sync_copy(px_hbm, px_v)
    pltpu.sync_copy(py_hbm, py_v)
    pltpu.sync_copy(pz_hbm, pz_v)
    pltpu.sync_copy(zi_hbm, zi_v)
    pltpu.sync_copy(bt_hbm, bt_v)
    pltpu.sync_copy(ainv_hbm, ainv_v)
    pltpu.sync_copy(st_hbm, st_v)
    pltpu.sync_copy(en_hbm, en_v)

    iota = lax.iota(jnp.int32, _L)
    row_lo = wid * _ROWS_PER_W
    row_hi = jnp.minimum(row_lo + _ROWS_PER_W, _N)

    def row_body(i, acc):
        b = bt_v[i]
        s = st_v[b]
        e = en_v[b]
        xi = px_v[i]
        yi = py_v[i]
        zi = pz_v[i]
        av = ainv_v[zi_v[i]]
        c0 = (s // _L) * _L
        nch = (e - c0 + (_L - 1)) // _L

        def chunk_body(k, acc_in):
            c = c0 + k * _L
            jv = c + iota
            dx = xi - px_v[pl.ds(c, _L)]
            dy = yi - py_v[pl.ds(c, _L)]
            dz = zi - pz_v[pl.ds(c, _L)]
            d2 = dx * dx + dy * dy + dz * dz
            msk = ((jv >= s) & (jv < e) & (jv != i)
                   & (d2 <= jnp.float32(_CUTOFF2)))
            d2c = jnp.maximum(d2, jnp.float32(1e-30))
            seed = jnp.int32(0x5F3759DF) - (plsc.bitcast(d2c, jnp.int32) >> 1)
            y = plsc.bitcast(seed, jnp.float32)
            h = jnp.float32(-0.5) * d2c
            for _ in range(3):
                y = y * (jnp.float32(1.5) + h * y * y)
            t = (d2c * y) * av
            f = (jnp.float32(0.1818) * jnp.exp(jnp.float32(-3.2) * t)
                 + jnp.float32(0.5099) * jnp.exp(jnp.float32(-0.9423) * t)
                 + jnp.float32(0.2802) * jnp.exp(jnp.float32(-0.4029) * t)
                 + jnp.float32(0.02817) * jnp.exp(jnp.float32(-0.2016) * t))
            return acc_in + jnp.where(msk, f * y, jnp.float32(0.0))

        return lax.fori_loop(0, nch, chunk_body, acc)

    acc = lax.fori_loop(row_lo, row_hi, row_body,
                        jnp.zeros((_L,), jnp.float32))
    acc_v[...] = acc
    pltpu.sync_copy(acc_v, out_hbm.at[wid])


def kernel(x, z, pos, batch, atomic_number):
    del x
    ps = pos.astype(jnp.float32) * jnp.float32(_DISTANCE_SCALE)
    px = ps[:, 0]
    py = ps[:, 1]
    pz = ps[:, 2]
    zi = z.astype(jnp.int32)
    bt = batch.astype(jnp.int32)
    ainv = (2.0 / 0.8854) * atomic_number.astype(jnp.float32) ** 0.23
    ainv_t = jnp.zeros((128,), jnp.float32).at[:_MAX_Z].set(ainv)
    ids = jnp.arange(_MAX_Z, dtype=jnp.int32)
    st = jnp.searchsorted(bt, ids, side="left").astype(jnp.int32)
    en = jnp.searchsorted(bt, ids, side="right").astype(jnp.int32)
    st_t = jnp.zeros((128,), jnp.int32).at[:_MAX_Z].set(st)
    en_t = jnp.zeros((128,), jnp.int32).at[:_MAX_Z].set(en)

    mesh = plsc.VectorSubcoreMesh(core_axis_name="c", subcore_axis_name="s",
                                  num_cores=_NC, num_subcores=_NS)
    partials = pl.kernel(
        _zbl_body,
        out_type=jax.ShapeDtypeStruct((_NW, _L), jnp.float32),
        mesh=mesh,
        scratch_types=[
            pltpu.VMEM((_N,), jnp.float32),
            pltpu.VMEM((_N,), jnp.float32),
            pltpu.VMEM((_N,), jnp.float32),
            pltpu.VMEM((_N,), jnp.int32),
            pltpu.VMEM((_N,), jnp.int32),
            pltpu.VMEM((128,), jnp.float32),
            pltpu.VMEM((128,), jnp.int32),
            pltpu.VMEM((128,), jnp.int32),
            pltpu.VMEM((_L,), jnp.float32),
        ],
    )(px, py, pz, zi, bt, ainv_t, st_t, en_t)
    return jnp.sum(partials) * jnp.float32(_OUT_SCALE)


# speedup vs baseline: 24.1387x; 24.1387x over previous
import jax
import jax.numpy as jnp
from jax import lax
from jax.experimental import pallas as pl
from jax.experimental.pallas import tpu as pltpu
from jax.experimental.pallas import tpu_sc as plsc

_MAX_Z = 100
_CUTOFF2 = 10.0 * 10.0
_DISTANCE_SCALE = 1e-10 * 18897300000.0
_ENERGY_SCALE = 1.602176634e-19
_OUT_SCALE = 2.30707755e-19 / _ENERGY_SCALE

_N = 10000
_NP = _N + 16
_NC = 2
_NS = 16
_NW = _NC * _NS
_L = 16
_ROWS_PER_W = (_N + _NW - 1) // _NW


def _zbl_body(px_hbm, py_hbm, pz_hbm, zi_hbm, bt_hbm, ainv_hbm, st_hbm,
              en_hbm, out_hbm, px_v, py_v, pz_v, zi_v, bt_v, ainv_v, st_v,
              en_v, acc_v):
    cid = lax.axis_index("c")
    sid = lax.axis_index("s")
    wid = sid * _NC + cid

    pltpu.sync_copy(px_hbm, px_v)
    pltpu.sync_copy(py_hbm, py_v)
    pltpu.sync_copy(pz_hbm, pz_v)
    pltpu.sync_copy(zi_hbm, zi_v)
    pltpu.sync_copy(bt_hbm, bt_v)
    pltpu.sync_copy(ainv_hbm, ainv_v)
    pltpu.sync_copy(st_hbm, st_v)
    pltpu.sync_copy(en_hbm, en_v)

    iota = lax.iota(jnp.int32, _L)
    row_lo = wid * _ROWS_PER_W
    row_hi = jnp.minimum(row_lo + _ROWS_PER_W, _N)

    def row_body(i, acc):
        b = bt_v[pl.ds(i, _L)][0]
        s = st_v[pl.ds(b, _L)][0]
        e = en_v[pl.ds(b, _L)][0]
        xi = px_v[pl.ds(i, _L)][0]
        yi = py_v[pl.ds(i, _L)][0]
        zi = pz_v[pl.ds(i, _L)][0]
        av = ainv_v[pl.ds(zi_v[pl.ds(i, _L)][0], _L)][0]
        c0 = (s // _L) * _L
        nch = (e - c0 + (_L - 1)) // _L

        def chunk_body(k, acc_in):
            c = c0 + k * _L
            jv = c + iota
            dx = xi - px_v[pl.ds(c, _L)]
            dy = yi - py_v[pl.ds(c, _L)]
            dz = zi - pz_v[pl.ds(c, _L)]
            d2 = dx * dx + dy * dy + dz * dz
            msk = ((jv >= s) & (jv < e) & (jv != i)
                   & (d2 <= jnp.float32(_CUTOFF2)))
            d2c = jnp.maximum(d2, jnp.float32(1e-30))
            seed = jnp.int32(0x5F3759DF) - (lax.bitcast_convert_type(d2c, jnp.int32) >> 1)
            y = lax.bitcast_convert_type(seed, jnp.float32)
            h = jnp.float32(-0.5) * d2c
            for _ in range(3):
                y = y * (jnp.float32(1.5) + h * y * y)
            t = (d2c * y) * av
            f = (jnp.float32(0.1818) * jnp.exp(jnp.float32(-3.2) * t)
                 + jnp.float32(0.5099) * jnp.exp(jnp.float32(-0.9423) * t)
                 + jnp.float32(0.2802) * jnp.exp(jnp.float32(-0.4029) * t)
                 + jnp.float32(0.02817) * jnp.exp(jnp.float32(-0.2016) * t))
            return acc_in + jnp.where(msk, f * y, jnp.float32(0.0))

        return lax.fori_loop(0, nch, chunk_body, acc)

    acc = lax.fori_loop(row_lo, row_hi, row_body,
                        jnp.zeros((_L,), jnp.float32))
    acc_v[...] = acc
    pltpu.sync_copy(acc_v, out_hbm.at[wid])


def kernel(x, z, pos, batch, atomic_number):
    del x
    ps = pos.astype(jnp.float32) * jnp.float32(_DISTANCE_SCALE)
    pad = _NP - _N
    px = jnp.pad(ps[:, 0], (0, pad))
    py = jnp.pad(ps[:, 1], (0, pad))
    pz = jnp.pad(ps[:, 2], (0, pad))
    zi = jnp.pad(z.astype(jnp.int32), (0, pad))
    bt = batch.astype(jnp.int32)
    bt_p = jnp.pad(bt, (0, pad))
    ainv = (2.0 / 0.8854) * atomic_number.astype(jnp.float32) ** 0.23
    ainv_t = jnp.zeros((128,), jnp.float32).at[:_MAX_Z].set(ainv)
    ids = jnp.arange(_MAX_Z, dtype=jnp.int32)
    st = jnp.searchsorted(bt, ids, side="left").astype(jnp.int32)
    en = jnp.searchsorted(bt, ids, side="right").astype(jnp.int32)
    st_t = jnp.zeros((128,), jnp.int32).at[:_MAX_Z].set(st)
    en_t = jnp.zeros((128,), jnp.int32).at[:_MAX_Z].set(en)

    mesh = plsc.VectorSubcoreMesh(core_axis_name="c", subcore_axis_name="s",
                                  num_cores=_NC, num_subcores=_NS)
    partials = pl.kernel(
        _zbl_body,
        out_type=jax.ShapeDtypeStruct((_NW, _L), jnp.float32),
        mesh=mesh,
        scratch_types=[
            pltpu.VMEM((_NP,), jnp.float32),
            pltpu.VMEM((_NP,), jnp.float32),
            pltpu.VMEM((_NP,), jnp.float32),
            pltpu.VMEM((_NP,), jnp.int32),
            pltpu.VMEM((_NP,), jnp.int32),
            pltpu.VMEM((128,), jnp.float32),
            pltpu.VMEM((128,), jnp.int32),
            pltpu.VMEM((128,), jnp.int32),
            pltpu.VMEM((_L,), jnp.float32),
        ],
    )(px, py, pz, zi, bt_p, ainv_t, st_t, en_t)
    return jnp.sum(partials) * jnp.float32(_OUT_SCALE)
